# Initial kernel scaffold; baseline (speedup 1.0000x reference)
import jax
import jax.numpy as jnp
from jax.experimental import pallas as pl


def kernel(point_bxyz):
    raise NotImplementedError("write your pallas kernel here")


# speedup vs baseline: 2.1071x; 2.1071x over previous
import jax
import jax.numpy as jnp
from jax import lax
from jax.experimental import pallas as pl
from jax.experimental.pallas import tpu as pltpu
from jax.experimental.pallas import tpu_sc as plsc

N = 400000
G = 64
BINS = 4 * G * G * G
HALF = BINS // 2

NW = 32
CHUNK = 128
CPT = 98
CPT1 = 2 * CPT
NPAD = NW * CPT * CHUNK
ROWS = NPAD // 128
HSIZE = HALF + 2048
HSLICE = HSIZE // 16
R = 1024
RPAD = 1032
FIRE = 14


def _hash_body(pts_ref, out_ref):
  v = pts_ref[...]
  shp = v.shape
  gf = jnp.float32(G)
  comp = lax.broadcasted_iota(jnp.int32, shp, 1) & 3
  m1 = comp == 1
  m2 = comp == 2
  m3 = comp == 3
  inf = jnp.float32(jnp.inf)
  xmin = jnp.min(jnp.where(m1, v, inf))
  xmax = jnp.max(jnp.where(m1, v, -inf))
  ymin = jnp.min(jnp.where(m2, v, inf))
  ymax = jnp.max(jnp.where(m2, v, -inf))
  zmin = jnp.min(jnp.where(m3, v, inf))
  zmax = jnp.max(jnp.where(m3, v, -inf))
  vx = (xmax - xmin) / gf
  vy = (ymax - ymin) / gf
  vz = (zmax - zmin) / gf
  minsel = jnp.where(m1, xmin, jnp.where(m2, ymin, jnp.where(m3, zmin, 0.0)))
  vssel = jnp.where(m1, vx, jnp.where(m2, vy, jnp.where(m3, vz, 1.0)))
  q = (v - minsel) / vssel
  cc = jnp.clip(jnp.floor(q).astype(jnp.int32), 0, G - 1)
  w = jnp.where(m1, jnp.float32(G * G), jnp.float32(1.0))
  w = jnp.where(m2, jnp.float32(G), w)
  w = jnp.where(comp == 0, jnp.float32(G * G * G), w)
  wc = cc.astype(jnp.float32) * w
  srow = lax.broadcasted_iota(jnp.int32, (512, 128), 0)
  scol = lax.broadcasted_iota(jnp.int32, (512, 128), 1)
  sel = (srow // 4 == scol).astype(jnp.float32)
  lin = jnp.dot(wc, sel, preferred_element_type=jnp.float32,
                precision=lax.Precision.HIGHEST).astype(jnp.int32)
  nr = N // 128
  pad = jnp.full((ROWS - nr, 128), BINS, jnp.int32)
  padh = jnp.full((ROWS - nr, 128), HALF, jnp.int32)
  out_ref[0:nr] = lin
  out_ref[nr:ROWS] = pad
  out_ref[ROWS:ROWS + nr] = jnp.where(lin < HALF, lin, HALF)
  out_ref[ROWS + nr:2 * ROWS] = padh
  out_ref[2 * ROWS:2 * ROWS + nr] = jnp.where(lin >= HALF, lin - HALF, HALF)
  out_ref[2 * ROWS + nr:3 * ROWS] = padh


def _hash_call(pts):
  return pl.pallas_call(
      _hash_body,
      out_shape=jax.ShapeDtypeStruct((3 * ROWS, 128), jnp.int32),
  )(pts)


def _prefix_body(lo_ref, hi_ref, out_ref):
  p = jnp.concatenate(
      [(lo_ref[...] > 0.0), (hi_ref[...] > 0.0)], axis=0).astype(jnp.float32)
  row = lax.broadcasted_iota(jnp.int32, (R, R), 0)
  col = lax.broadcasted_iota(jnp.int32, (R, R), 1)
  u_strict = (row < col).astype(jnp.bfloat16)
  within = jnp.dot(p.astype(jnp.bfloat16), u_strict,
                   preferred_element_type=jnp.float32)
  rowsum = jnp.sum(p, axis=1, keepdims=True)
  l_strict = (col < row).astype(jnp.float32)
  before = jnp.dot(l_strict, rowsum, preferred_element_type=jnp.float32)
  out_ref[0:R] = (before + within).astype(jnp.int32)
  out_ref[R:RPAD] = jnp.zeros((RPAD - R, R), jnp.int32)


def _prefix_call(cnt_lo, cnt_hi):
  return pl.pallas_call(
      _prefix_body,
      out_shape=jax.ShapeDtypeStruct((RPAD, R), jnp.int32),
  )(cnt_lo, cnt_hi)


def _count_body(lin_ref, ones_ref, zeros_ref, out_ref, idx_v, ones_v, zbuf_v,
                hist, sem):
  s = lax.axis_index("s")

  pltpu.sync_copy(ones_ref, ones_v)
  pltpu.sync_copy(zeros_ref, zbuf_v)
  pltpu.sync_copy(zbuf_v, hist.at[pl.ds(s * HSLICE, HSLICE)])
  pltpu.sync_copy(lin_ref.at[s], idx_v)
  plsc.subcore_barrier()

  def scatter_group(g, carry):
    descs = []
    for j in range(FIRE):
      ch = g * FIRE + j
      descs.append(
          pltpu.async_copy(ones_v, hist.at[idx_v.at[ch]], sem, add=True))
    for d in descs:
      d.wait()
    return carry

  lax.fori_loop(0, CPT1 // FIRE, scatter_group, 0)
  plsc.subcore_barrier()

  pltpu.sync_copy(hist.at[pl.ds(s * HSLICE, HSLICE)], zbuf_v)
  pltpu.sync_copy(zbuf_v, out_ref.at[pl.ds(s * HSLICE, HSLICE)])


def _count_call(lin_s, ones_c, zeros_c):
  mesh = plsc.VectorSubcoreMesh(
      core_axis_name="c", subcore_axis_name="s", num_cores=1)
  return pl.kernel(
      _count_body,
      out_type=jax.ShapeDtypeStruct((HSIZE,), jnp.float32),
      mesh=mesh,
      scratch_types=[
          pltpu.VMEM((CPT1, CHUNK), jnp.int32),
          pltpu.VMEM((CHUNK,), jnp.float32),
          pltpu.VMEM((HSLICE,), jnp.float32),
          pltpu.VMEM_SHARED((HSIZE,), jnp.float32),
          pltpu.SemaphoreType.DMA,
      ],
  )(lin_s, ones_c, zeros_c)


def _gather_body(lin_ref, rank_ref, out_ref, idx_v, res_v, sem):
  c = lax.axis_index("c")
  s = lax.axis_index("s")
  wid = c * 16 + s
  pltpu.sync_copy(lin_ref.at[wid], idx_v)

  def gather_group(g, carry):
    descs = []
    for j in range(FIRE):
      ch = g * FIRE + j
      descs.append(
          pltpu.async_copy(rank_ref.at[idx_v.at[ch]], res_v.at[ch], sem))
    for d in descs:
      d.wait()
    return carry

  lax.fori_loop(0, CPT // FIRE, gather_group, 0)
  pltpu.sync_copy(res_v, out_ref.at[wid])


def _gather_call(lin_t, ranks_flat):
  mesh = plsc.VectorSubcoreMesh(core_axis_name="c", subcore_axis_name="s")
  return pl.kernel(
      _gather_body,
      out_type=jax.ShapeDtypeStruct((NW, CPT, CHUNK), jnp.int32),
      mesh=mesh,
      scratch_types=[
          pltpu.VMEM((CPT, CHUNK), jnp.int32),
          pltpu.VMEM((CPT, CHUNK), jnp.int32),
          pltpu.SemaphoreType.DMA,
      ],
  )(lin_t, ranks_flat)


@jax.jit
def kernel(point_bxyz):
  pts = point_bxyz.reshape(N // 128, 512)
  hashed = _hash_call(pts)
  lin_t = hashed[0:ROWS].reshape(NW, CPT, CHUNK)
  lo_s = hashed[ROWS:2 * ROWS].reshape(16, CPT1, CHUNK)
  hi_s = hashed[2 * ROWS:3 * ROWS].reshape(16, CPT1, CHUNK)
  ones_c = jnp.ones((CHUNK,), jnp.float32)
  zeros_c = jnp.zeros((HSLICE,), jnp.float32)
  cnt_lo = _count_call(lo_s, ones_c, zeros_c)[:HALF].reshape(R // 2, R)
  cnt_hi = _count_call(hi_s, ones_c, zeros_c)[:HALF].reshape(R // 2, R)
  ranks = _prefix_call(cnt_lo, cnt_hi)
  ranks_flat = ranks.reshape(-1)
  gids = _gather_call(lin_t, ranks_flat)
  return gids.reshape(-1)[:N]


# speedup vs baseline: 2.1142x; 1.0034x over previous
import jax
import jax.numpy as jnp
from jax import lax
from jax.experimental import pallas as pl
from jax.experimental.pallas import tpu as pltpu
from jax.experimental.pallas import tpu_sc as plsc

N = 400000
G = 64
BINS = 4 * G * G * G
HALF = BINS // 2

NW = 32
CHUNK = 128
CPT = 98
CPT1 = 2 * CPT
NPAD = NW * CPT * CHUNK
ROWS = NPAD // 128
HSIZE = HALF + 2048
HSLICE = HSIZE // 16
R = 1024
RPAD = 1032
FIRE = 14


def _hash_body(pts_ref, out_ref, lo_ref, hi_ref):
  v = pts_ref[...]
  shp = v.shape
  gf = jnp.float32(G)
  comp = lax.broadcasted_iota(jnp.int32, shp, 1) & 3
  m1 = comp == 1
  m2 = comp == 2
  m3 = comp == 3
  inf = jnp.float32(jnp.inf)
  xmin = jnp.min(jnp.where(m1, v, inf))
  xmax = jnp.max(jnp.where(m1, v, -inf))
  ymin = jnp.min(jnp.where(m2, v, inf))
  ymax = jnp.max(jnp.where(m2, v, -inf))
  zmin = jnp.min(jnp.where(m3, v, inf))
  zmax = jnp.max(jnp.where(m3, v, -inf))
  vx = (xmax - xmin) / gf
  vy = (ymax - ymin) / gf
  vz = (zmax - zmin) / gf
  minsel = jnp.where(m1, xmin, jnp.where(m2, ymin, jnp.where(m3, zmin, 0.0)))
  vssel = jnp.where(m1, vx, jnp.where(m2, vy, jnp.where(m3, vz, 1.0)))
  q = (v - minsel) / vssel
  cc = jnp.clip(jnp.floor(q).astype(jnp.int32), 0, G - 1)
  w = jnp.where(m1, jnp.float32(G * G), jnp.float32(1.0))
  w = jnp.where(m2, jnp.float32(G), w)
  w = jnp.where(comp == 0, jnp.float32(G * G * G), w)
  wc = cc.astype(jnp.float32) * w
  srow = lax.broadcasted_iota(jnp.int32, (512, 128), 0)
  scol = lax.broadcasted_iota(jnp.int32, (512, 128), 1)
  sel = (srow // 4 == scol).astype(jnp.float32)
  lin = jnp.dot(wc, sel, preferred_element_type=jnp.float32,
                precision=lax.Precision.HIGHEST).astype(jnp.int32)
  nr = N // 128
  pad = jnp.full((ROWS - nr, 128), BINS, jnp.int32)
  padh = jnp.full((ROWS - nr, 128), HALF, jnp.int32)
  lin_full = jnp.concatenate([lin, pad], axis=0)
  lo_full = jnp.concatenate([jnp.where(lin < HALF, lin, HALF), padh], axis=0)
  hi_full = jnp.concatenate(
      [jnp.where(lin >= HALF, lin - HALF, HALF), padh], axis=0)
  out_ref[...] = lin_full.reshape(NW, CPT, CHUNK)
  lo_ref[...] = lo_full.reshape(16, CPT1, CHUNK)
  hi_ref[...] = hi_full.reshape(16, CPT1, CHUNK)


def _hash_call(pts):
  return pl.pallas_call(
      _hash_body,
      out_shape=[
          jax.ShapeDtypeStruct((NW, CPT, CHUNK), jnp.int32),
          jax.ShapeDtypeStruct((16, CPT1, CHUNK), jnp.int32),
          jax.ShapeDtypeStruct((16, CPT1, CHUNK), jnp.int32),
      ],
  )(pts)


def _prefix_body(lo_ref, hi_ref, out_ref):
  p = jnp.concatenate(
      [(lo_ref[...] > 0.0), (hi_ref[...] > 0.0)], axis=0).astype(jnp.float32)
  row = lax.broadcasted_iota(jnp.int32, (R, R), 0)
  col = lax.broadcasted_iota(jnp.int32, (R, R), 1)
  u_strict = (row < col).astype(jnp.bfloat16)
  within = jnp.dot(p.astype(jnp.bfloat16), u_strict,
                   preferred_element_type=jnp.float32)
  rowsum = jnp.sum(p, axis=1, keepdims=True)
  l_strict = (col < row).astype(jnp.float32)
  before = jnp.dot(l_strict, rowsum, preferred_element_type=jnp.float32)
  out_ref[0:R] = (before + within).astype(jnp.int32)
  out_ref[R:RPAD] = jnp.zeros((RPAD - R, R), jnp.int32)


def _prefix_call(cnt_lo, cnt_hi):
  return pl.pallas_call(
      _prefix_body,
      out_shape=jax.ShapeDtypeStruct((RPAD, R), jnp.int32),
  )(cnt_lo, cnt_hi)


def _count_body(lin_ref, ones_ref, zeros_ref, out_ref, idx_v, ones_v, zbuf_v,
                hist, sem):
  s = lax.axis_index("s")

  pltpu.sync_copy(ones_ref, ones_v)
  pltpu.sync_copy(zeros_ref, zbuf_v)
  pltpu.sync_copy(zbuf_v, hist.at[pl.ds(s * HSLICE, HSLICE)])
  pltpu.sync_copy(lin_ref.at[s], idx_v)
  plsc.subcore_barrier()

  def scatter_group(g, carry):
    descs = []
    for j in range(FIRE):
      ch = g * FIRE + j
      descs.append(
          pltpu.async_copy(ones_v, hist.at[idx_v.at[ch]], sem, add=True))
    for d in descs:
      d.wait()
    return carry

  lax.fori_loop(0, CPT1 // FIRE, scatter_group, 0)
  plsc.subcore_barrier()

  pltpu.sync_copy(hist.at[pl.ds(s * HSLICE, HSLICE)], zbuf_v)
  pltpu.sync_copy(zbuf_v, out_ref.at[pl.ds(s * HSLICE, HSLICE)])


def _count_call(lin_s, ones_c, zeros_c):
  mesh = plsc.VectorSubcoreMesh(
      core_axis_name="c", subcore_axis_name="s", num_cores=1)
  return pl.kernel(
      _count_body,
      out_type=jax.ShapeDtypeStruct((HSIZE,), jnp.float32),
      mesh=mesh,
      scratch_types=[
          pltpu.VMEM((CPT1, CHUNK), jnp.int32),
          pltpu.VMEM((CHUNK,), jnp.float32),
          pltpu.VMEM((HSLICE,), jnp.float32),
          pltpu.VMEM_SHARED((HSIZE,), jnp.float32),
          pltpu.SemaphoreType.DMA,
      ],
  )(lin_s, ones_c, zeros_c)


def _gather_body(lin_ref, rank_ref, out_ref, idx_v, res_v, sem):
  c = lax.axis_index("c")
  s = lax.axis_index("s")
  wid = c * 16 + s
  pltpu.sync_copy(lin_ref.at[wid], idx_v)

  def gather_group(g, carry):
    descs = []
    for j in range(FIRE):
      ch = g * FIRE + j
      descs.append(
          pltpu.async_copy(rank_ref.at[idx_v.at[ch]], res_v.at[ch], sem))
    for d in descs:
      d.wait()
    return carry

  lax.fori_loop(0, CPT // FIRE, gather_group, 0)
  pltpu.sync_copy(res_v, out_ref.at[wid])


def _gather_call(lin_t, ranks_flat):
  mesh = plsc.VectorSubcoreMesh(core_axis_name="c", subcore_axis_name="s")
  return pl.kernel(
      _gather_body,
      out_type=jax.ShapeDtypeStruct((NW, CPT, CHUNK), jnp.int32),
      mesh=mesh,
      scratch_types=[
          pltpu.VMEM((CPT, CHUNK), jnp.int32),
          pltpu.VMEM((CPT, CHUNK), jnp.int32),
          pltpu.SemaphoreType.DMA,
      ],
  )(lin_t, ranks_flat)


@jax.jit
def kernel(point_bxyz):
  pts = point_bxyz.reshape(N // 128, 512)
  lin_t, lo_s, hi_s = _hash_call(pts)
  ones_c = jnp.ones((CHUNK,), jnp.float32)
  zeros_c = jnp.zeros((HSLICE,), jnp.float32)
  cnt_lo = _count_call(lo_s, ones_c, zeros_c)[:HALF].reshape(R // 2, R)
  cnt_hi = _count_call(hi_s, ones_c, zeros_c)[:HALF].reshape(R // 2, R)
  ranks = _prefix_call(cnt_lo, cnt_hi)
  ranks_flat = ranks.reshape(-1)
  gids = _gather_call(lin_t, ranks_flat)
  return gids.reshape(-1)[:N]


# speedup vs baseline: 2.6684x; 1.2621x over previous
import jax
import jax.numpy as jnp
from jax import lax
from jax.experimental import pallas as pl
from jax.experimental.pallas import tpu as pltpu
from jax.experimental.pallas import tpu_sc as plsc

N = 400000
G = 64
BINS = 4 * G * G * G
HALF = BINS // 2

NW = 32
CHUNK = 128
CPT = 98
CPT1 = 2 * CPT
NPAD = NW * CPT * CHUNK
ROWS = NPAD // 128
HSIZE = HALF + 2048
HSLICE = HSIZE // 16
R = 1024
RPAD = 1032
FIRE = 14


def _hash_body(pts_ref, out_ref, lohi_ref):
  v = pts_ref[...]
  shp = v.shape
  gf = jnp.float32(G)
  comp = lax.broadcasted_iota(jnp.int32, shp, 1) & 3
  m1 = comp == 1
  m2 = comp == 2
  m3 = comp == 3
  inf = jnp.float32(jnp.inf)
  xmin = jnp.min(jnp.where(m1, v, inf))
  xmax = jnp.max(jnp.where(m1, v, -inf))
  ymin = jnp.min(jnp.where(m2, v, inf))
  ymax = jnp.max(jnp.where(m2, v, -inf))
  zmin = jnp.min(jnp.where(m3, v, inf))
  zmax = jnp.max(jnp.where(m3, v, -inf))
  vx = (xmax - xmin) / gf
  vy = (ymax - ymin) / gf
  vz = (zmax - zmin) / gf
  minsel = jnp.where(m1, xmin, jnp.where(m2, ymin, jnp.where(m3, zmin, 0.0)))
  vssel = jnp.where(m1, vx, jnp.where(m2, vy, jnp.where(m3, vz, 1.0)))
  q = (v - minsel) / vssel
  cc = jnp.clip(jnp.floor(q).astype(jnp.int32), 0, G - 1)
  w = jnp.where(m1, jnp.float32(G * G), jnp.float32(1.0))
  w = jnp.where(m2, jnp.float32(G), w)
  w = jnp.where(comp == 0, jnp.float32(G * G * G), w)
  wc = cc.astype(jnp.float32) * w
  srow = lax.broadcasted_iota(jnp.int32, (512, 128), 0)
  scol = lax.broadcasted_iota(jnp.int32, (512, 128), 1)
  sel = (srow // 4 == scol).astype(jnp.float32)
  lin = jnp.dot(wc, sel, preferred_element_type=jnp.float32,
                precision=lax.Precision.HIGHEST).astype(jnp.int32)
  nr = N // 128
  pad = jnp.full((ROWS - nr, 128), BINS, jnp.int32)
  lin_full = jnp.concatenate([lin, pad], axis=0)
  out_ref[...] = lin_full.reshape(NW, CPT, CHUNK)
  lohi_ref[0:16] = jnp.where(
      lin_full < HALF, lin_full, HALF).reshape(16, CPT1, CHUNK)
  lohi_ref[16:32] = jnp.where(
      lin_full >= HALF, lin_full - HALF, HALF).reshape(16, CPT1, CHUNK)


def _hash_call(pts):
  return pl.pallas_call(
      _hash_body,
      out_shape=[
          jax.ShapeDtypeStruct((NW, CPT, CHUNK), jnp.int32),
          jax.ShapeDtypeStruct((NW, CPT1, CHUNK), jnp.int32),
      ],
  )(pts)


def _prefix_body(lo_ref, hi_ref, out_ref):
  p = jnp.concatenate(
      [(lo_ref[...] > 0.0), (hi_ref[...] > 0.0)], axis=0).astype(jnp.float32)
  row = lax.broadcasted_iota(jnp.int32, (R, R), 0)
  col = lax.broadcasted_iota(jnp.int32, (R, R), 1)
  u_strict = (row < col).astype(jnp.bfloat16)
  within = jnp.dot(p.astype(jnp.bfloat16), u_strict,
                   preferred_element_type=jnp.float32)
  rowsum = jnp.sum(p, axis=1, keepdims=True)
  l_strict = (col < row).astype(jnp.float32)
  before = jnp.dot(l_strict, rowsum, preferred_element_type=jnp.float32)
  out_ref[0:R] = (before + within).astype(jnp.int32)
  out_ref[R:RPAD] = jnp.zeros((RPAD - R, R), jnp.int32)


def _prefix_call(cnt_lo, cnt_hi):
  return pl.pallas_call(
      _prefix_body,
      out_shape=jax.ShapeDtypeStruct((RPAD, R), jnp.int32),
  )(cnt_lo, cnt_hi)


def _count_body(lin_ref, ones_ref, zeros_ref, out_ref, idx_v, ones_v, zbuf_v,
                hist, sem):
  c = lax.axis_index("c")
  s = lax.axis_index("s")

  pltpu.sync_copy(ones_ref, ones_v)
  pltpu.sync_copy(zeros_ref, zbuf_v)
  pltpu.sync_copy(zbuf_v, hist.at[pl.ds(s * HSLICE, HSLICE)])
  pltpu.sync_copy(lin_ref.at[c * 16 + s], idx_v)
  plsc.subcore_barrier()

  def scatter_group(g, carry):
    descs = []
    for j in range(FIRE):
      ch = g * FIRE + j
      descs.append(
          pltpu.async_copy(ones_v, hist.at[idx_v.at[ch]], sem, add=True))
    for d in descs:
      d.wait()
    return carry

  lax.fori_loop(0, CPT1 // FIRE, scatter_group, 0)
  plsc.subcore_barrier()

  pltpu.sync_copy(hist.at[pl.ds(s * HSLICE, HSLICE)], zbuf_v)
  pltpu.sync_copy(zbuf_v, out_ref.at[pl.ds(c * HSIZE + s * HSLICE, HSLICE)])


def _count_call(lin_s, ones_c, zeros_c):
  mesh = plsc.VectorSubcoreMesh(core_axis_name="c", subcore_axis_name="s")
  return pl.kernel(
      _count_body,
      out_type=jax.ShapeDtypeStruct((2 * HSIZE,), jnp.float32),
      mesh=mesh,
      scratch_types=[
          pltpu.VMEM((CPT1, CHUNK), jnp.int32),
          pltpu.VMEM((CHUNK,), jnp.float32),
          pltpu.VMEM((HSLICE,), jnp.float32),
          pltpu.VMEM_SHARED((HSIZE,), jnp.float32),
          pltpu.SemaphoreType.DMA,
      ],
  )(lin_s, ones_c, zeros_c)


def _gather_body(lin_ref, rank_ref, out_ref, idx_v, res_v, sem):
  c = lax.axis_index("c")
  s = lax.axis_index("s")
  wid = c * 16 + s
  pltpu.sync_copy(lin_ref.at[wid], idx_v)

  def gather_group(g, carry):
    descs = []
    for j in range(FIRE):
      ch = g * FIRE + j
      descs.append(
          pltpu.async_copy(rank_ref.at[idx_v.at[ch]], res_v.at[ch], sem))
    for d in descs:
      d.wait()
    return carry

  lax.fori_loop(0, CPT // FIRE, gather_group, 0)
  pltpu.sync_copy(res_v, out_ref.at[wid])


def _gather_call(lin_t, ranks_flat):
  mesh = plsc.VectorSubcoreMesh(core_axis_name="c", subcore_axis_name="s")
  return pl.kernel(
      _gather_body,
      out_type=jax.ShapeDtypeStruct((NW, CPT, CHUNK), jnp.int32),
      mesh=mesh,
      scratch_types=[
          pltpu.VMEM((CPT, CHUNK), jnp.int32),
          pltpu.VMEM((CPT, CHUNK), jnp.int32),
          pltpu.SemaphoreType.DMA,
      ],
  )(lin_t, ranks_flat)


@jax.jit
def kernel(point_bxyz):
  pts = point_bxyz.reshape(N // 128, 512)
  lin_t, lohi = _hash_call(pts)
  ones_c = jnp.ones((CHUNK,), jnp.float32)
  zeros_c = jnp.zeros((HSLICE,), jnp.float32)
  counts = _count_call(lohi, ones_c, zeros_c)
  cnt_lo = counts[0:HALF].reshape(R // 2, R)
  cnt_hi = counts[HSIZE:HSIZE + HALF].reshape(R // 2, R)
  ranks = _prefix_call(cnt_lo, cnt_hi)
  ranks_flat = ranks.reshape(-1)
  gids = _gather_call(lin_t, ranks_flat)
  return gids.reshape(-1)[:N]
